# baseline (speedup 1.0000x reference)
import functools

import jax
import jax.numpy as jnp
from jax import lax
from jax.experimental import pallas as pl
from jax.experimental.pallas import tpu as pltpu
from jax.experimental.pallas import tpu_sc as plsc


def _matvec_body(e_ref, w_ref, s_ref):
    s_ref[0] = lax.dot_general(
        w_ref[...], e_ref[...], (((1,), (0,)), ((), ())),
        preferred_element_type=jnp.float32,
    )


def _table_dot(emb, W1, col0=0, Vc=32768):
    V, D = emb.shape
    embT = emb.T
    assert col0 % Vc == 0
    blk0 = col0 // Vc
    grid = pl.cdiv(V - col0, Vc)
    s2d = pl.pallas_call(
        _matvec_body,
        grid=(grid,),
        in_specs=[
            pl.BlockSpec((D, Vc), lambda i: (0, i + blk0)),
            pl.BlockSpec((1, D), lambda i: (0, 0)),
        ],
        out_specs=pl.BlockSpec((1, 1, Vc), lambda i: (i, 0, 0)),
        out_shape=jax.ShapeDtypeStruct((grid, 1, Vc), jnp.float32),
    )(embT, W1)
    return s2d.reshape(-1)


def _make_sc_matvec(VT, D, NW, NC, PW, CHC):
    mesh = plsc.VectorSubcoreMesh(core_axis_name="c", subcore_axis_name="s")
    NCHUNK = PW // CHC

    @functools.partial(
        pl.kernel,
        out_type=jax.ShapeDtypeStruct((NW, PW), jnp.float32),
        mesh=mesh,
        scratch_types=[
            pltpu.VMEM((D, CHC), jnp.float32),
            pltpu.VMEM((D, 16), jnp.float32),
            pltpu.VMEM((PW,), jnp.float32),
        ],
    )
    def sc_matvec(embT_hbm, wbc_hbm, out_hbm, ebuf, wbuf, acc):
        wid = lax.axis_index("s") * NC + lax.axis_index("c")
        base = VT + wid * PW
        pltpu.sync_copy(wbc_hbm, wbuf)

        def chunk(c, carry):
            pltpu.sync_copy(embT_hbm.at[:, pl.ds(base + c * CHC, CHC)], ebuf)
            for dblk in range(D // 8):
                wv = [wbuf[dblk * 8 + k, :] for k in range(8)]

                def g_body(g, carry2):
                    sl = pl.ds(g * 16, 16)
                    asl = pl.ds(c * CHC + g * 16, 16)
                    part = ebuf[dblk * 8, sl] * wv[0]
                    for k in range(1, 8):
                        part = part + ebuf[dblk * 8 + k, sl] * wv[k]
                    if dblk == 0:
                        acc[asl] = part
                    else:
                        acc[asl] = acc[asl] + part
                    return carry2

                lax.fori_loop(0, CHC // 16, g_body, 0)
            return carry

        lax.fori_loop(0, NCHUNK, chunk, 0)
        pltpu.sync_copy(acc, out_hbm.at[wid])

    return sc_matvec


def _make_gather(V, NW, NCH, CH, NC):
    mesh = plsc.VectorSubcoreMesh(core_axis_name="c", subcore_axis_name="s")

    @functools.partial(
        pl.kernel,
        out_type=jax.ShapeDtypeStruct((NW, NCH, CH), jnp.float32),
        mesh=mesh,
        scratch_types=[
            pltpu.VMEM((NCH, CH), jnp.int32),
            pltpu.VMEM((NCH, CH), jnp.float32),
            pltpu.SemaphoreType.DMA,
        ],
    )
    def gather_k(s_hbm, x_hbm, out_hbm, idx_v, rows_v, sem):
        wid = lax.axis_index("s") * NC + lax.axis_index("c")
        pltpu.sync_copy(x_hbm.at[wid], idx_v)

        def body(j, carry):
            pltpu.async_copy(s_hbm.at[idx_v.at[j]], rows_v.at[j], sem)
            return carry

        lax.fori_loop(0, NCH, body, 0)
        pltpu.make_async_copy(out_hbm.at[wid], rows_v, sem).wait()
        pltpu.sync_copy(rows_v, out_hbm.at[wid])

    return gather_k


def _head_body(rt_ref, w2_ref, b1_ref, o_ref):
    rt = rt_ref[...]
    w2 = w2_ref[...]
    logits = lax.dot_general(
        w2, rt, (((1,), (0,)), ((), ())),
        preferred_element_type=jnp.float32,
    )
    bias = b1_ref[0, 0] * jnp.sum(w2, axis=1)
    logits = logits + bias[:, None]
    m = jnp.max(logits, axis=0, keepdims=True)
    lse = m + jnp.log(jnp.sum(jnp.exp(logits - m), axis=0, keepdims=True))
    o_ref[...] = logits - lse


def _head(rt, W2, b1):
    L, B = rt.shape
    C = W2.shape[0]
    return pl.pallas_call(
        _head_body,
        in_specs=[
            pl.BlockSpec((L, B), lambda: (0, 0)),
            pl.BlockSpec((C, L), lambda: (0, 0)),
            pl.BlockSpec((1, 1), lambda: (0, 0)),
        ],
        out_specs=pl.BlockSpec((C, B), lambda: (0, 0)),
        out_shape=jax.ShapeDtypeStruct((C, B), jnp.float32),
    )(rt, W2, b1.reshape(1, 1))


def kernel(x, emb, W1, b1, W2):
    B, L = x.shape
    V, D = emb.shape

    info = plsc.get_sparse_core_info()
    NC, NS = info.num_cores, info.num_subcores
    NW = NC * NS

    PW = 7168
    VSC = NW * PW
    s_hi = _table_dot(emb, W1, col0=VSC)
    wbc = jnp.broadcast_to(W1.reshape(D, 1), (D, 16))
    s_lo = _make_sc_matvec(0, D, NW, NC, PW, 512)(emb.T, wbc)
    s = jnp.concatenate([s_lo.reshape(-1), s_hi[: V - VSC]])
    CH = 128
    total = B * L
    NCH = total // (NW * CH)
    assert total == NW * NCH * CH

    xr = x.T.reshape(NW, NCH, CH)
    r = _make_gather(V, NW, NCH, CH, NC)(s, xr)

    return _head(r.reshape(L, B), W2, b1).T


# speedup vs baseline: 1.0851x; 1.0851x over previous
import functools

import jax
import jax.numpy as jnp
from jax import lax
from jax.experimental import pallas as pl
from jax.experimental.pallas import tpu as pltpu
from jax.experimental.pallas import tpu_sc as plsc


def _matvec_body(e_ref, w_ref, s_ref):
    s_ref[0] = lax.dot_general(
        w_ref[...], e_ref[...], (((1,), (0,)), ((), ())),
        preferred_element_type=jnp.float32,
    )


def _table_dot(emb, W1, Vc=16384):
    V, D = emb.shape
    embT = emb.T
    grid = pl.cdiv(V, Vc)
    s2d = pl.pallas_call(
        _matvec_body,
        grid=(grid,),
        in_specs=[
            pl.BlockSpec((D, Vc), lambda i: (0, i)),
            pl.BlockSpec((1, D), lambda i: (0, 0)),
        ],
        out_specs=pl.BlockSpec((1, 1, Vc), lambda i: (i, 0, 0)),
        out_shape=jax.ShapeDtypeStruct((grid, 1, Vc), jnp.float32),
    )(embT, W1)
    return s2d.reshape(-1)


def _make_gather(V, NW, NCH, CH, NC):
    mesh = plsc.VectorSubcoreMesh(core_axis_name="c", subcore_axis_name="s")

    @functools.partial(
        pl.kernel,
        out_type=jax.ShapeDtypeStruct((NW, NCH, CH), jnp.float32),
        mesh=mesh,
        scratch_types=[
            pltpu.VMEM((NCH, CH), jnp.int32),
            pltpu.VMEM((NCH, CH), jnp.float32),
            pltpu.SemaphoreType.DMA,
        ],
    )
    def gather_k(s_hbm, x_hbm, out_hbm, idx_v, rows_v, sem):
        wid = lax.axis_index("s") * NC + lax.axis_index("c")
        pltpu.sync_copy(x_hbm.at[wid], idx_v)

        def body(j, carry):
            pltpu.async_copy(s_hbm.at[idx_v.at[j]], rows_v.at[j], sem)
            return carry

        lax.fori_loop(0, NCH, body, 0)
        pltpu.make_async_copy(out_hbm.at[wid], rows_v, sem).wait()
        pltpu.sync_copy(rows_v, out_hbm.at[wid])

    return gather_k


def _head_body(rt_ref, w2_ref, b1_ref, o_ref):
    rt = rt_ref[...]
    w2 = w2_ref[...]
    logits = lax.dot_general(
        w2, rt, (((1,), (0,)), ((), ())),
        preferred_element_type=jnp.float32,
    )
    bias = b1_ref[0, 0] * jnp.sum(w2, axis=1)
    logits = logits + bias[:, None]
    m = jnp.max(logits, axis=0, keepdims=True)
    lse = m + jnp.log(jnp.sum(jnp.exp(logits - m), axis=0, keepdims=True))
    o_ref[...] = logits - lse


def _head(rt, W2, b1):
    L, B = rt.shape
    C = W2.shape[0]
    return pl.pallas_call(
        _head_body,
        in_specs=[
            pl.BlockSpec((L, B), lambda: (0, 0)),
            pl.BlockSpec((C, L), lambda: (0, 0)),
            pl.BlockSpec((1, 1), lambda: (0, 0)),
        ],
        out_specs=pl.BlockSpec((C, B), lambda: (0, 0)),
        out_shape=jax.ShapeDtypeStruct((C, B), jnp.float32),
    )(rt, W2, b1.reshape(1, 1))


def kernel(x, emb, W1, b1, W2):
    B, L = x.shape
    V, D = emb.shape

    s = _table_dot(emb, W1)

    info = plsc.get_sparse_core_info()
    NC, NS = info.num_cores, info.num_subcores
    NW = NC * NS
    CH = 128
    total = B * L
    NCH = total // (NW * CH)
    assert total == NW * NCH * CH

    xr = x.T.reshape(NW, NCH, CH)
    r = _make_gather(V, NW, NCH, CH, NC)(s, xr)

    return _head(r.reshape(L, B), W2, b1).T


# speedup vs baseline: 1.3191x; 1.2156x over previous
import functools

import jax
import jax.numpy as jnp
from jax import lax
from jax.experimental import pallas as pl
from jax.experimental.pallas import tpu as pltpu
from jax.experimental.pallas import tpu_sc as plsc


def _matvec_body(e_ref, w_ref, s_ref):
    s_ref[0] = lax.dot_general(
        w_ref[...], e_ref[...], (((1,), (0,)), ((), ())),
        preferred_element_type=jnp.float32,
    )


def _table_dot(emb, W1, Vc=32768):
    V, D = emb.shape
    embT = emb.T
    grid = pl.cdiv(V, Vc)
    s2d = pl.pallas_call(
        _matvec_body,
        grid=(grid,),
        in_specs=[
            pl.BlockSpec((D, Vc), lambda i: (0, i)),
            pl.BlockSpec((1, D), lambda i: (0, 0)),
        ],
        out_specs=pl.BlockSpec((1, 1, Vc), lambda i: (i, 0, 0)),
        out_shape=jax.ShapeDtypeStruct((grid, 1, Vc), jnp.float32),
    )(embT, W1)
    return s2d.reshape(-1)


def _make_gather(SV, NW, NCH, CH, NC, NS):
    mesh = plsc.VectorSubcoreMesh(core_axis_name="c", subcore_axis_name="s")
    stage = SV // NS

    @functools.partial(
        pl.kernel,
        out_type=jax.ShapeDtypeStruct((NW, NCH, CH), jnp.float32),
        mesh=mesh,
        scratch_types=[
            pltpu.VMEM((NCH, CH), jnp.int32),
            pltpu.VMEM((NCH, CH), jnp.float32),
            pltpu.VMEM_SHARED((SV,), jnp.float32),
            pltpu.SemaphoreType.DMA,
        ],
    )
    def gather_k(s_hbm, x_hbm, out_hbm, idx_v, rows_v, s_sh, sem):
        cid = lax.axis_index("c")
        sid = lax.axis_index("s")
        wid = sid * NC + cid
        pltpu.sync_copy(
            s_hbm.at[pl.ds(sid * stage, stage)],
            s_sh.at[pl.ds(sid * stage, stage)],
        )
        pltpu.sync_copy(x_hbm.at[wid], idx_v)
        plsc.subcore_barrier()

        def body(j, carry):
            pltpu.async_copy(s_sh.at[idx_v.at[j]], rows_v.at[j], sem)
            return carry

        lax.fori_loop(0, NCH, body, 0)
        pltpu.make_async_copy(out_hbm.at[wid], rows_v, sem).wait()
        pltpu.sync_copy(rows_v, out_hbm.at[wid])

    return gather_k


def _head_body(rt_ref, w2_ref, b1_ref, o_ref):
    rt = rt_ref[...]
    w2 = w2_ref[...]
    logits = lax.dot_general(
        w2, rt, (((1,), (0,)), ((), ())),
        preferred_element_type=jnp.float32,
    )
    bias = b1_ref[0, 0] * jnp.sum(w2, axis=1)
    logits = logits + bias[:, None]
    m = jnp.max(logits, axis=0, keepdims=True)
    lse = m + jnp.log(jnp.sum(jnp.exp(logits - m), axis=0, keepdims=True))
    o_ref[...] = logits - lse


def _head(rt, W2, b1):
    L, B = rt.shape
    C = W2.shape[0]
    return pl.pallas_call(
        _head_body,
        in_specs=[
            pl.BlockSpec((L, B), lambda: (0, 0)),
            pl.BlockSpec((C, L), lambda: (0, 0)),
            pl.BlockSpec((1, 1), lambda: (0, 0)),
        ],
        out_specs=pl.BlockSpec((C, B), lambda: (0, 0)),
        out_shape=jax.ShapeDtypeStruct((C, B), jnp.float32),
    )(rt, W2, b1.reshape(1, 1))


def kernel(x, emb, W1, b1, W2):
    B, L = x.shape
    V, D = emb.shape

    s = _table_dot(emb, W1)

    info = plsc.get_sparse_core_info()
    NC, NS = info.num_cores, info.num_subcores
    NW = NC * NS
    CH = 128
    total = B * L
    NCH = total // (NW * CH)
    assert total == NW * NCH * CH

    xr = x.T.reshape(NW, NCH, CH)
    r = _make_gather(s.shape[0], NW, NCH, CH, NC, NS)(s, xr)

    return _head(r.reshape(L, B), W2, b1).T
